# baseline (speedup 1.0000x reference)
import functools

import jax
import jax.numpy as jnp
from jax import lax
from jax.experimental import pallas as pl
from jax.experimental.pallas import tpu as pltpu
from jax.experimental.pallas import tpu_sc as plsc

_NC, _NS, _L = 2, 16, 16
_CP = 256


@functools.lru_cache(maxsize=None)
def _make_sc(B, S, V):
    cols_per_w = B // _NS
    slots = cols_per_w * _CP
    n_chunks = -(-S // _L)
    half = V // _NC
    n_out = cols_per_w // _L

    mesh = plsc.VectorSubcoreMesh(
        core_axis_name="c", subcore_axis_name="s",
        num_cores=_NC, num_subcores=_NS)

    @functools.partial(
        pl.kernel,
        out_type=jax.ShapeDtypeStruct((_NC, B), jnp.float32),
        mesh=mesh,
        scratch_types=[
            pltpu.VMEM((slots,), jnp.int32),
            pltpu.VMEM((half,), jnp.float32),
            pltpu.VMEM((half,), jnp.int32),
            pltpu.VMEM((cols_per_w,), jnp.float32),
            pltpu.SemaphoreType.DMA,
        ],
        compiler_params=pltpu.CompilerParams(needs_layout_passes=False),
    )
    def sc(text_hbm, w_hbm, out_hbm, tok_v, wch_v, tag_v, out_v, sem):
        cid = lax.axis_index("c")
        sid = lax.axis_index("s")
        lo = cid * half
        pltpu.sync_copy(text_hbm.at[sid], tok_v)
        wdesc = pltpu.async_copy(w_hbm.at[pl.ds(lo, half)], wch_v, sem)

        lane = lax.iota(jnp.int32, _L)
        poss = [lane + c * _L for c in range(n_chunks)]
        uhalf = jnp.uint32(half)

        def scatter(col):
            tvecs, masks = [], []
            base = col * _CP
            for c in range(n_chunks):
                idx = tok_v[pl.ds(base + c * _L, _L)]
                t = idx - lo
                inr = t.astype(jnp.uint32) < uhalf
                if (c + 1) * _L > S:
                    inr = jnp.logical_and(inr, poss[c] < S)
                tvecs.append(t)
                masks.append(inr)
                plsc.store_scatter(tag_v, [t], poss[c], mask=inr)
            return tvecs, masks

        def readback(tvecs, masks):
            sels = []
            for c in range(n_chunks):
                tags = plsc.load_gather(tag_v, [tvecs[c]], mask=masks[c])
                sels.append(jnp.logical_and(masks[c], tags == poss[c]))
            return sels

        def accumulate(col, tvecs, sels, outs):
            accs = [jnp.zeros((_L,), jnp.float32) for _ in range(4)]
            for c in range(n_chunks):
                wv = plsc.load_gather(wch_v, [tvecs[c]], mask=sels[c])
                accs[c % 4] = accs[c % 4] + jnp.where(sels[c], wv,
                                                      jnp.float32(0))
            s = jnp.sum((accs[0] + accs[1]) + (accs[2] + accs[3]))
            return tuple(
                jnp.where(lane == col - k * _L, outs[k] + s, outs[k])
                for k in range(n_out)
            )

        def pair_work(ca, cb, outs, before_accumulate=None):
            ta, ma = scatter(ca)
            sa = readback(ta, ma)
            tb, mb = scatter(cb)
            if before_accumulate is not None:
                before_accumulate()
            outs = accumulate(ca, ta, sa, outs)
            sb = readback(tb, mb)
            outs = accumulate(cb, tb, sb, outs)
            return outs

        def pair_body(i, outs):
            return pair_work(2 * i, 2 * i + 1, outs)

        outs = tuple(jnp.zeros((_L,), jnp.float32) for _ in range(n_out))
        outs = pair_work(0, 1, outs, before_accumulate=wdesc.wait)
        outs = lax.fori_loop(1, cols_per_w // 2, pair_body, outs)

        for k in range(n_out):
            out_v[pl.ds(k * _L, _L)] = outs[k]
        pltpu.sync_copy(out_v,
                        out_hbm.at[cid, pl.ds(sid * cols_per_w, cols_per_w)])

    return sc


def kernel(text, W, b):
    S, B = text.shape
    V = W.shape[1]
    t = jnp.pad(text.T.astype(jnp.int32), ((0, 0), (0, _CP - S)))
    t2 = t.reshape(_NS, (B // _NS) * _CP)
    parts = _make_sc(B, S, V)(t2, W.reshape(-1))
    return (parts[0] + parts[1] + b).reshape(B, 1)


# speedup vs baseline: 1.0065x; 1.0065x over previous
import functools

import jax
import jax.numpy as jnp
from jax import lax
from jax.experimental import pallas as pl
from jax.experimental.pallas import tpu as pltpu
from jax.experimental.pallas import tpu_sc as plsc

_NC, _NS, _L = 2, 16, 16
_CP = 256


@functools.lru_cache(maxsize=None)
def _make_sc(B, S, V):
    cols_per_w = B // _NS
    slots = cols_per_w * _CP
    n_chunks = -(-S // _L)
    half = V // _NC
    n_out = cols_per_w // _L

    mesh = plsc.VectorSubcoreMesh(
        core_axis_name="c", subcore_axis_name="s",
        num_cores=_NC, num_subcores=_NS)

    @functools.partial(
        pl.kernel,
        out_type=jax.ShapeDtypeStruct((_NC, B), jnp.float32),
        mesh=mesh,
        scratch_types=[
            pltpu.VMEM((slots,), jnp.int32),
            pltpu.VMEM((half,), jnp.float32),
            pltpu.VMEM((half,), jnp.int32),
            pltpu.VMEM((cols_per_w,), jnp.float32),
        ],
        compiler_params=pltpu.CompilerParams(needs_layout_passes=False),
    )
    def sc(text_hbm, w_hbm, out_hbm, tok_v, wch_v, tag_v, out_v):
        cid = lax.axis_index("c")
        sid = lax.axis_index("s")
        lo = cid * half
        pltpu.sync_copy(text_hbm.at[sid], tok_v)
        pltpu.sync_copy(w_hbm.at[pl.ds(lo, half)], wch_v)

        lane = lax.iota(jnp.int32, _L)
        poss = [lane + c * _L for c in range(n_chunks)]
        uhalf = jnp.uint32(half)

        def scatter(col):
            tvecs, masks = [], []
            base = col * _CP
            for c in range(n_chunks):
                idx = tok_v[pl.ds(base + c * _L, _L)]
                t = idx - lo
                inr = t.astype(jnp.uint32) < uhalf
                if (c + 1) * _L > S:
                    inr = jnp.logical_and(inr, poss[c] < S)
                tvecs.append(t)
                masks.append(inr)
                plsc.store_scatter(tag_v, [t], poss[c], mask=inr)
            return tvecs, masks

        def readback(tvecs, masks):
            sels = []
            for c in range(n_chunks):
                tags = plsc.load_gather(tag_v, [tvecs[c]], mask=masks[c])
                sels.append(jnp.logical_and(masks[c], tags == poss[c]))
            return sels

        def accumulate(col, tvecs, sels, outs):
            accs = [jnp.zeros((_L,), jnp.float32) for _ in range(4)]
            for c in range(n_chunks):
                wv = plsc.load_gather(wch_v, [tvecs[c]], mask=sels[c])
                accs[c % 4] = accs[c % 4] + jnp.where(sels[c], wv,
                                                      jnp.float32(0))
            s = jnp.sum((accs[0] + accs[1]) + (accs[2] + accs[3]))
            return tuple(
                jnp.where(lane == col - k * _L, outs[k] + s, outs[k])
                for k in range(n_out)
            )

        def pair_body(i, outs):
            ca, cb = 2 * i, 2 * i + 1
            ta, ma = scatter(ca)
            sa = readback(ta, ma)
            tb, mb = scatter(cb)
            outs = accumulate(ca, ta, sa, outs)
            sb = readback(tb, mb)
            outs = accumulate(cb, tb, sb, outs)
            return outs

        outs = tuple(jnp.zeros((_L,), jnp.float32) for _ in range(n_out))
        outs = lax.fori_loop(0, cols_per_w // 2, pair_body, outs)

        for k in range(n_out):
            out_v[pl.ds(k * _L, _L)] = outs[k]
        pltpu.sync_copy(out_v,
                        out_hbm.at[cid, pl.ds(sid * cols_per_w, cols_per_w)])

    return sc


def kernel(text, W, b):
    S, B = text.shape
    V = W.shape[1]
    t = jnp.pad(text.T.astype(jnp.int32), ((0, 0), (0, _CP - S)))
    t2 = t.reshape(_NS, (B // _NS) * _CP)
    parts = _make_sc(B, S, V)(t2, W.reshape(-1))
    return (parts[0] + parts[1] + b).reshape(B, 1)
